# baseline (speedup 1.0000x reference)
import functools

import jax
import jax.numpy as jnp
from jax import lax
from jax.experimental import pallas as pl
from jax.experimental.pallas import tpu as pltpu
from jax.experimental.pallas import tpu_sc as plsc

_B_BLK = 8
_NC, _NS = 2, 16
_NW = _NC * _NS


def _code_body(x_ref, cb_ref, code_ref):
    S = x_ref.shape[2]
    kiota = jax.lax.broadcasted_iota(jnp.int32, (32, S), 0)

    c0 = cb_ref[0]
    c1 = cb_ref[1]
    z = jnp.zeros((32, 32), jnp.float32)
    cblk = jnp.concatenate(
        [jnp.concatenate([c0, z], axis=1),
         jnp.concatenate([z, c1], axis=1)], axis=0)
    cnorm = jnp.sum(cblk * cblk, axis=1, keepdims=True)
    cs = -2.0 * cblk

    for b in range(_B_BLK):
        x = x_ref[b]
        dotT = jax.lax.dot_general(
            cs, x, (((1,), (0,)), ((), ())),
            precision=jax.lax.Precision.HIGHEST,
            preferred_element_type=jnp.float32)
        score = cnorm + dotT

        inds = []
        for h in range(2):
            s = score[h * 32:(h + 1) * 32]
            mins = jnp.min(s, axis=0, keepdims=True)
            ind = jnp.min(jnp.where(s <= mins, kiota, 32), axis=0)
            inds.append(ind)
        code_ref[b, :] = inds[0] + 32 * inds[1]


def _blend_body(x_hbm, code_hbm, tbl_hbm, al_hbm, out_hbm,
                xbuf, codebuf, indbuf, tblbuf, albuf):
    wid = lax.axis_index("s") * _NC + lax.axis_index("c")
    pltpu.sync_copy(tbl_hbm, tblbuf)
    pltpu.sync_copy(al_hbm, albuf)
    al = albuf[...]
    be = 1.0 - al

    for t in range(4):
        slab = wid * 4 + t
        b = slab // 2
        half = slab % 2
        sh = 5 * half
        tbase = half * 1024

        pltpu.sync_copy(code_hbm.at[b], codebuf)

        def prep(ic, _, sh=sh):
            for u in range(4):
                o = (ic * 4 + u) * 16
                c16 = codebuf[pl.ds(o, 16)]
                indbuf[pl.ds(o, 16)] = (
                    lax.shift_right_logical(c16, sh) & 31)
            return 0
        lax.fori_loop(0, 16, prep, 0)

        pltpu.sync_copy(x_hbm.at[b, pl.ds(half * 32, 32)], xbuf)

        def drow(d, _, tbase=tbase):
            base = tbase + d * 32

            def chunk(ic, _, base=base, d=d):
                for u in range(4):
                    o = (ic * 4 + u) * 16
                    ind = indbuf[pl.ds(o, 16)]
                    cvals = plsc.load_gather(tblbuf, [ind + base])
                    xv = xbuf[d, pl.ds(o, 16)]
                    xbuf[d, pl.ds(o, 16)] = al * xv + be * cvals
                return 0
            lax.fori_loop(0, 16, chunk, 0)
            return 0
        lax.fori_loop(0, 32, drow, 0)

        pltpu.sync_copy(xbuf, out_hbm.at[b, pl.ds(half * 32, 32)])


@functools.partial(jax.jit, static_argnames=())
def kernel(input, kernel, alpha):
    B, S, D = input.shape
    xt = jnp.transpose(input, (0, 2, 1))
    alpha_f = jnp.asarray(alpha, jnp.float32)

    code = pl.pallas_call(
        _code_body,
        grid=(B // _B_BLK,),
        in_specs=[
            pl.BlockSpec((_B_BLK, D, S), lambda i: (i, 0, 0)),
            pl.BlockSpec((2, 32, 32), lambda i: (0, 0, 0)),
        ],
        out_specs=pl.BlockSpec((_B_BLK, S), lambda i: (i, 0)),
        out_shape=jax.ShapeDtypeStruct((B, S), jnp.int32),
    )(xt, kernel)

    tbl = jnp.transpose(kernel, (0, 2, 1)).reshape(-1)
    alvec = jnp.full((16,), alpha_f, jnp.float32)

    blend = functools.partial(
        pl.kernel,
        mesh=plsc.VectorSubcoreMesh(core_axis_name="c", subcore_axis_name="s"),
        compiler_params=pltpu.CompilerParams(needs_layout_passes=False),
        out_type=jax.ShapeDtypeStruct((B, D, S), jnp.float32),
        scratch_types=[
            pltpu.VMEM((32, S), jnp.float32),
            pltpu.VMEM((S,), jnp.int32),
            pltpu.VMEM((S,), jnp.int32),
            pltpu.VMEM((2048,), jnp.float32),
            pltpu.VMEM((16,), jnp.float32),
        ],
    )(_blend_body)
    out_t = blend(xt, code, tbl, alvec)
    return jnp.transpose(out_t, (0, 2, 1)), code


# speedup vs baseline: 2.1512x; 2.1512x over previous
import functools

import jax
import jax.numpy as jnp
from jax import lax
from jax.experimental import pallas as pl
from jax.experimental.pallas import tpu as pltpu
from jax.experimental.pallas import tpu_sc as plsc

_B_BLK = 8
_NC, _NS = 2, 16
_NW = _NC * _NS


def _code_body(x_ref, cb_ref, code_ref):
    S = x_ref.shape[2]
    kiota = jax.lax.broadcasted_iota(jnp.int32, (32, S), 0)

    c0 = cb_ref[0]
    c1 = cb_ref[1]
    z = jnp.zeros((32, 32), jnp.float32)
    cblk = jnp.concatenate(
        [jnp.concatenate([c0, z], axis=1),
         jnp.concatenate([z, c1], axis=1)], axis=0)
    cnorm = jnp.sum(cblk * cblk, axis=1, keepdims=True)
    cs = -2.0 * cblk

    for b in range(_B_BLK):
        x = x_ref[b]
        dotT = jax.lax.dot_general(
            cs, x, (((1,), (0,)), ((), ())),
            precision=jax.lax.Precision.HIGHEST,
            preferred_element_type=jnp.float32)
        score = cnorm + dotT

        inds = []
        for h in range(2):
            s = score[h * 32:(h + 1) * 32]
            mins = jnp.min(s, axis=0, keepdims=True)
            ind = jnp.min(jnp.where(s <= mins, kiota, 32), axis=0)
            inds.append(ind)
        code_ref[b, :] = inds[0] + 32 * inds[1]


def _blend_body(x_hbm, code_hbm, tbl_hbm, al_hbm, out_hbm,
                xbuf, codebuf, indbuf, tblbuf, albuf):
    wid = lax.axis_index("s") * _NC + lax.axis_index("c")
    pltpu.sync_copy(tbl_hbm, tblbuf)
    pltpu.sync_copy(al_hbm, albuf)
    al = albuf[...]
    be = 1.0 - al

    for t in range(4):
        slab = wid * 4 + t
        b = slab // 2
        half = slab % 2
        sh = 5 * half
        tbase = half * 1024

        pltpu.sync_copy(code_hbm.at[b], codebuf)

        @plsc.parallel_loop(0, 64, unroll=8)
        def _prep(i, sh=sh):
            c16 = codebuf[pl.ds(i * 16, 16)]
            indbuf[pl.ds(i * 16, 16)] = lax.shift_right_logical(c16, sh) & 31

        pltpu.sync_copy(x_hbm.at[b, pl.ds(half * 32, 32)], xbuf)

        @plsc.parallel_loop(0, 32 * 64, unroll=8)
        def _chunk(q, tbase=tbase):
            d = q >> 6
            o = (q & 63) * 16
            ind = indbuf[pl.ds(o, 16)]
            cvals = plsc.load_gather(tblbuf, [ind + (tbase + d * 32)])
            xv = xbuf[d, pl.ds(o, 16)]
            xbuf[d, pl.ds(o, 16)] = al * xv + be * cvals

        pltpu.sync_copy(xbuf, out_hbm.at[b, pl.ds(half * 32, 32)])


@functools.partial(jax.jit, static_argnames=())
def kernel(input, kernel, alpha):
    B, S, D = input.shape
    xt = jnp.transpose(input, (0, 2, 1))
    alpha_f = jnp.asarray(alpha, jnp.float32)

    code = pl.pallas_call(
        _code_body,
        grid=(B // _B_BLK,),
        in_specs=[
            pl.BlockSpec((_B_BLK, D, S), lambda i: (i, 0, 0)),
            pl.BlockSpec((2, 32, 32), lambda i: (0, 0, 0)),
        ],
        out_specs=pl.BlockSpec((_B_BLK, S), lambda i: (i, 0)),
        out_shape=jax.ShapeDtypeStruct((B, S), jnp.int32),
    )(xt, kernel)

    tbl = jnp.transpose(kernel, (0, 2, 1)).reshape(-1)
    alvec = jnp.full((16,), alpha_f, jnp.float32)

    blend = functools.partial(
        pl.kernel,
        mesh=plsc.VectorSubcoreMesh(core_axis_name="c", subcore_axis_name="s"),
        compiler_params=pltpu.CompilerParams(needs_layout_passes=False),
        out_type=jax.ShapeDtypeStruct((B, D, S), jnp.float32),
        scratch_types=[
            pltpu.VMEM((32, S), jnp.float32),
            pltpu.VMEM((S,), jnp.int32),
            pltpu.VMEM((S,), jnp.int32),
            pltpu.VMEM((2048,), jnp.float32),
            pltpu.VMEM((16,), jnp.float32),
        ],
    )(_blend_body)
    out_t = blend(xt, code, tbl, alvec)
    return jnp.transpose(out_t, (0, 2, 1)), code


# speedup vs baseline: 2.3225x; 1.0796x over previous
import functools

import jax
import jax.numpy as jnp
from jax import lax
from jax.experimental import pallas as pl
from jax.experimental.pallas import tpu as pltpu
from jax.experimental.pallas import tpu_sc as plsc

_B_BLK = 8
_NC, _NS = 2, 16
_NW = _NC * _NS


def _code_body(x_ref, cb_ref, code_ref):
    S = x_ref.shape[2]
    kiota = jax.lax.broadcasted_iota(jnp.int32, (32, S), 0)

    c0 = cb_ref[0]
    c1 = cb_ref[1]
    z = jnp.zeros((32, 32), jnp.float32)
    cblk = jnp.concatenate(
        [jnp.concatenate([c0, z], axis=1),
         jnp.concatenate([z, c1], axis=1)], axis=0)
    cnorm = jnp.sum(cblk * cblk, axis=1, keepdims=True)
    cs = -2.0 * cblk

    for b in range(_B_BLK):
        x = x_ref[b]
        dotT = jax.lax.dot_general(
            cs, x, (((1,), (0,)), ((), ())),
            precision=jax.lax.Precision.HIGHEST,
            preferred_element_type=jnp.float32)
        score = cnorm + dotT

        inds = []
        for h in range(2):
            s = score[h * 32:(h + 1) * 32]
            mins = jnp.min(s, axis=0, keepdims=True)
            ind = jnp.min(jnp.where(s <= mins, kiota, 32), axis=0)
            inds.append(ind)
        code_ref[b, :] = inds[0] + 32 * inds[1]


def _blend_body(x_hbm, code_hbm, tbl_hbm, al_hbm, out_hbm,
                xbuf, codebuf, indbuf, tblbuf, albuf, insems, outsems):
    wid = lax.axis_index("s") * _NC + lax.axis_index("c")
    pltpu.sync_copy(tbl_hbm, tblbuf)
    pltpu.sync_copy(al_hbm, albuf)
    al = albuf[...]
    be = 1.0 - al

    nslab = 4

    def slab_src(t):
        slab = wid * nslab + t
        b = slab // 2
        half = slab % 2
        return b, half

    def start_in(t):
        b, half = slab_src(t)
        return pltpu.async_copy(
            x_hbm.at[b, pl.ds(half * 32, 32)], xbuf.at[t % 2],
            insems.at[t % 2])

    in_h = {0: start_in(0)}
    out_h = {}
    for t in range(nslab):
        b, half = slab_src(t)
        if t + 1 < nslab:
            if t >= 1:
                out_h[t - 1].wait()
            in_h[t + 1] = start_in(t + 1)

        pltpu.sync_copy(code_hbm.at[b], codebuf)
        sh = 5 * half
        tbase = half * 1024

        @plsc.parallel_loop(0, 64, unroll=8)
        def _prep(i, sh=sh):
            c16 = codebuf[pl.ds(i * 16, 16)]
            indbuf[pl.ds(i * 16, 16)] = lax.shift_right_logical(c16, sh) & 31

        in_h[t].wait()
        buf = t % 2

        @plsc.parallel_loop(0, 32 * 64, unroll=8)
        def _chunk(q, tbase=tbase, buf=buf):
            d = q >> 6
            o = (q & 63) * 16
            ind = indbuf[pl.ds(o, 16)]
            cvals = plsc.load_gather(tblbuf, [ind + (tbase + d * 32)])
            xv = xbuf[buf, d, pl.ds(o, 16)]
            xbuf[buf, d, pl.ds(o, 16)] = al * xv + be * cvals

        out_h[t] = pltpu.async_copy(
            xbuf.at[buf], out_hbm.at[b, pl.ds(half * 32, 32)],
            outsems.at[buf])

    out_h[nslab - 2].wait()
    out_h[nslab - 1].wait()


@functools.partial(jax.jit, static_argnames=())
def kernel(input, kernel, alpha):
    B, S, D = input.shape
    xt = jnp.transpose(input, (0, 2, 1))
    alpha_f = jnp.asarray(alpha, jnp.float32)

    code = pl.pallas_call(
        _code_body,
        grid=(B // _B_BLK,),
        in_specs=[
            pl.BlockSpec((_B_BLK, D, S), lambda i: (i, 0, 0)),
            pl.BlockSpec((2, 32, 32), lambda i: (0, 0, 0)),
        ],
        out_specs=pl.BlockSpec((_B_BLK, S), lambda i: (i, 0)),
        out_shape=jax.ShapeDtypeStruct((B, S), jnp.int32),
    )(xt, kernel)

    tbl = jnp.transpose(kernel, (0, 2, 1)).reshape(-1)
    alvec = jnp.full((16,), alpha_f, jnp.float32)

    blend = functools.partial(
        pl.kernel,
        mesh=plsc.VectorSubcoreMesh(core_axis_name="c", subcore_axis_name="s"),
        compiler_params=pltpu.CompilerParams(needs_layout_passes=False),
        out_type=jax.ShapeDtypeStruct((B, D, S), jnp.float32),
        scratch_types=[
            pltpu.VMEM((2, 32, S), jnp.float32),
            pltpu.VMEM((S,), jnp.int32),
            pltpu.VMEM((S,), jnp.int32),
            pltpu.VMEM((2048,), jnp.float32),
            pltpu.VMEM((16,), jnp.float32),
            pltpu.SemaphoreType.DMA((2,)),
            pltpu.SemaphoreType.DMA((2,)),
        ],
    )(_blend_body)
    out_t = blend(xt, code, tbl, alvec)
    return jnp.transpose(out_t, (0, 2, 1)), code
